# Initial kernel scaffold; baseline (speedup 1.0000x reference)
import jax
import jax.numpy as jnp
from jax.experimental import pallas as pl


def kernel(relative_position_bias_table, relative_position_index):
    raise NotImplementedError("write your pallas kernel here")


# speedup vs baseline: 4.0798x; 4.0798x over previous
import functools

import jax
import jax.numpy as jnp
from jax import lax
from jax.experimental import pallas as pl
from jax.experimental.pallas import tpu as pltpu
from jax.experimental.pallas import tpu_sc as plsc

WS = 32
NTOK = WS * WS + 1
N2 = NTOK * NTOK
NH = 16
NRD = (2 * WS - 1) ** 2 + 3
LANES = 16
F = NH * N2
C = 10000
NB = F // C
GPB = C // LANES
CW = C + 8
N2P = 1060624
NC = 2
NS = 16
NW = NC * NS


@functools.partial(
    pl.kernel,
    out_type=jax.ShapeDtypeStruct((F,), jnp.float32),
    mesh=plsc.VectorSubcoreMesh(core_axis_name="c", subcore_axis_name="s"),
    compiler_params=pltpu.CompilerParams(needs_layout_passes=False),
    scratch_types=[
        pltpu.VMEM((NRD * NH,), jnp.float32),
        pltpu.VMEM((CW,), jnp.int32),
        pltpu.VMEM((C,), jnp.int32),
        pltpu.VMEM((C,), jnp.float32),
    ],
)
def _rpb_kernel(table_hbm, idxp_hbm, out_hbm, table_v, idx1_v, idx2_v, out_v):
    wid = lax.axis_index("s") * NC + lax.axis_index("c")
    pltpu.sync_copy(table_hbm, table_v)

    def block(i, carry):
        b = wid + i * NW
        s = b * C
        h1 = s // N2
        p_lo = s - h1 * N2
        sh = lax.rem(p_lo, 8)
        a1 = p_lo - sh
        pltpu.sync_copy(idxp_hbm.at[pl.ds(pl.multiple_of(a1, 8), CW)], idx1_v)
        h1v = jnp.full((LANES,), 0, jnp.int32) + h1

        def pass1(k, carry):
            rows = idx1_v[pl.ds(sh + k * LANES, LANES)]
            out_v[pl.ds(k * LANES, LANES)] = plsc.load_gather(table_v,
                                                              [rows + h1v])
            return carry

        lax.fori_loop(0, GPB, pass1, 0, unroll=False)

        e_m = N2 - p_lo

        @pl.when(e_m < C)
        def _crossing():
            pltpu.sync_copy(idxp_hbm.at[pl.ds(0, C)], idx2_v)
            h2v = h1v + 1

            def pass2(k, carry):
                e = k * LANES + lax.iota(jnp.int32, LANES)
                msk = e >= e_m
                p2 = jnp.maximum(e - e_m, 0)
                rows = plsc.load_gather(idx2_v, [p2], mask=msk)
                vals = plsc.load_gather(table_v, [rows + h2v], mask=msk)
                plsc.store_scatter(out_v, [e], vals, mask=msk)
                return carry

            lax.fori_loop(e_m // LANES, GPB, pass2, 0, unroll=False)

        pltpu.sync_copy(out_v, out_hbm.at[pl.ds(pl.multiple_of(s, 8), C)])
        return carry

    nb_w = (NB - wid + NW - 1) // NW
    lax.fori_loop(0, nb_w, block, 0, unroll=False)


def kernel(relative_position_bias_table, relative_position_index):
    idx_flat = relative_position_index.reshape(-1) * NH
    idx_pad = jnp.zeros((N2P,), jnp.int32).at[:N2].set(idx_flat)
    out = _rpb_kernel(relative_position_bias_table.reshape(-1), idx_pad)
    return out.reshape(NH, NTOK, NTOK)


# speedup vs baseline: 4.5265x; 1.1095x over previous
import functools

import jax
import jax.numpy as jnp
from jax import lax
from jax.experimental import pallas as pl
from jax.experimental.pallas import tpu as pltpu
from jax.experimental.pallas import tpu_sc as plsc

WS = 32
NTOK = WS * WS + 1
N2 = NTOK * NTOK
NH = 16
NRD = (2 * WS - 1) ** 2 + 3
LANES = 16
F = NH * N2
C = 10000
NB = F // C
GPB = C // LANES
CW = C + 8
N2P = 1060624
NC = 2
NS = 16
NW = NC * NS


@functools.partial(
    pl.kernel,
    out_type=jax.ShapeDtypeStruct((F,), jnp.float32),
    mesh=plsc.VectorSubcoreMesh(core_axis_name="c", subcore_axis_name="s"),
    compiler_params=pltpu.CompilerParams(needs_layout_passes=False),
    scratch_types=[
        pltpu.VMEM((NRD * NH,), jnp.float32),
        pltpu.VMEM((CW,), jnp.int32),
        pltpu.VMEM((C,), jnp.int32),
        pltpu.VMEM((C,), jnp.float32),
    ],
)
def _rpb_kernel(table_hbm, idxp_hbm, out_hbm, table_v, idx1_v, idx2_v, out_v):
    wid = lax.axis_index("s") * NC + lax.axis_index("c")
    pltpu.sync_copy(table_hbm, table_v)

    def block(i, carry):
        b = wid + i * NW
        s = b * C
        h1 = s // N2
        p_lo = s - h1 * N2
        sh = lax.rem(p_lo, 8)
        a1 = p_lo - sh
        pltpu.sync_copy(idxp_hbm.at[pl.ds(pl.multiple_of(a1, 8), CW)], idx1_v)
        h1v = jnp.full((LANES,), 0, jnp.int32) + h1 * NRD

        def pass1(k, carry):
            rows = idx1_v[pl.ds(sh + k * LANES, LANES)]
            out_v[pl.ds(k * LANES, LANES)] = plsc.load_gather(table_v,
                                                              [rows + h1v])
            return carry

        lax.fori_loop(0, GPB, pass1, 0, unroll=False)

        e_m = N2 - p_lo

        @pl.when(e_m < C)
        def _crossing():
            pltpu.sync_copy(idxp_hbm.at[pl.ds(0, C)], idx2_v)
            h2v = h1v + NRD

            def pass2(k, carry):
                e = k * LANES + lax.iota(jnp.int32, LANES)
                msk = e >= e_m
                p2 = jnp.maximum(e - e_m, 0)
                rows = plsc.load_gather(idx2_v, [p2], mask=msk)
                vals = plsc.load_gather(table_v, [rows + h2v], mask=msk)
                plsc.store_scatter(out_v, [e], vals, mask=msk)
                return carry

            lax.fori_loop(e_m // LANES, GPB, pass2, 0, unroll=False)

        pltpu.sync_copy(out_v, out_hbm.at[pl.ds(pl.multiple_of(s, 8), C)])
        return carry

    nb_w = (NB - wid + NW - 1) // NW
    lax.fori_loop(0, nb_w, block, 0, unroll=False)


def kernel(relative_position_bias_table, relative_position_index):
    idx_flat = relative_position_index.reshape(-1)
    idx_pad = jnp.zeros((N2P,), jnp.int32).at[:N2].set(idx_flat)
    out = _rpb_kernel(relative_position_bias_table.T.reshape(-1), idx_pad)
    return out.reshape(NH, NTOK, NTOK)


# speedup vs baseline: 4.7096x; 1.0404x over previous
import functools

import jax
import jax.numpy as jnp
from jax import lax
from jax.experimental import pallas as pl
from jax.experimental.pallas import tpu as pltpu
from jax.experimental.pallas import tpu_sc as plsc

WS = 32
NTOK = WS * WS + 1
N2 = NTOK * NTOK
NH = 16
NRD = (2 * WS - 1) ** 2 + 3
LANES = 16
F = NH * N2
C = 5125
NCH = N2 // C
KG = 321
S_FULL = KG * LANES
S_LAST = 5128
W = 5152
PADN = 1050656
NC = 2
NS = 16
NW = NC * NS
HHALF = NH // 2
NU = NCH * 2


@functools.partial(
    pl.kernel,
    out_type=jax.ShapeDtypeStruct((F,), jnp.float32),
    mesh=plsc.VectorSubcoreMesh(core_axis_name="c", subcore_axis_name="s"),
    compiler_params=pltpu.CompilerParams(needs_layout_passes=False),
    scratch_types=[
        pltpu.VMEM((NRD * NH,), jnp.float32),
        pltpu.VMEM((W,), jnp.int32),
        pltpu.VMEM((S_FULL,), jnp.float32),
    ],
)
def _rpb_kernel(table_hbm, idxp_hbm, out_hbm, table_v, win_v, out_v):
    wid = lax.axis_index("s") * NC + lax.axis_index("c")
    pltpu.sync_copy(table_hbm, table_v)

    def unit(i, carry):
        u = wid + i * NW
        c = u // 2
        h0 = (u % 2) * HHALF
        p0 = c * C
        qb = (p0 // 8) * 8
        pltpu.sync_copy(idxp_hbm.at[pl.ds(pl.multiple_of(qb, 8), W)], win_v)

        def head(hh, carry2):
            h = h0 + hh
            o = h * N2 + p0
            shift = lax.rem(h + p0, 8)
            w0 = o - shift
            off0 = p0 - qb + 8 - shift
            base = h * NRD

            t0 = lax.iota(jnp.int32, LANES)
            bv0 = jnp.maximum(
                base - NRD * (t0 < (shift - p0)).astype(jnp.int32), 0)
            rows0 = win_v[pl.ds(off0, LANES)]
            out_v[pl.ds(0, LANES)] = plsc.load_gather(table_v, [rows0 + bv0])

            def mid(k, carry3):
                rows = win_v[pl.ds(off0 + k * LANES, LANES)]
                out_v[pl.ds(k * LANES, LANES)] = plsc.load_gather(
                    table_v, [rows + base])
                return carry3

            lax.fori_loop(1, KG - 1, mid, 0, unroll=False)

            tl = (KG - 1) * LANES + t0
            bvl = jnp.minimum(
                base + NRD * (tl >= (N2 - p0 + shift)).astype(jnp.int32),
                (NH - 1) * NRD)
            rowsl = win_v[pl.ds(off0 + (KG - 1) * LANES, LANES)]
            out_v[pl.ds((KG - 1) * LANES, LANES)] = plsc.load_gather(
                table_v, [rowsl + bvl])

            is_last = jnp.logical_and(c == NCH - 1, h == NH - 1)

            @pl.when(jnp.logical_not(is_last))
            def _full():
                pltpu.sync_copy(
                    out_v.at[pl.ds(0, S_FULL)],
                    out_hbm.at[pl.ds(pl.multiple_of(w0, 8), S_FULL)])

            @pl.when(is_last)
            def _last():
                pltpu.sync_copy(
                    out_v.at[pl.ds(0, S_LAST)],
                    out_hbm.at[pl.ds(pl.multiple_of(w0, 8), S_LAST)])

            return carry2

        lax.fori_loop(0, HHALF, head, 0, unroll=False)
        return carry

    nu_w = (NU - wid + NW - 1) // NW
    lax.fori_loop(0, nu_w, unit, 0, unroll=False)


def kernel(relative_position_bias_table, relative_position_index):
    idx_flat = relative_position_index.reshape(-1)
    idx_pad = (
        jnp.zeros((PADN,), jnp.int32)
        .at[8:8 + N2].set(idx_flat)
        .at[0:8].set(idx_flat[N2 - 8:])
        .at[8 + N2:8 + N2 + 16].set(idx_flat[:16])
    )
    out = _rpb_kernel(relative_position_bias_table.T.reshape(-1), idx_pad)
    return out.reshape(NH, NTOK, NTOK)


# speedup vs baseline: 5.0374x; 1.0696x over previous
import functools

import jax
import jax.numpy as jnp
from jax import lax
from jax.experimental import pallas as pl
from jax.experimental.pallas import tpu as pltpu
from jax.experimental.pallas import tpu_sc as plsc

WS = 32
NTOK = WS * WS + 1
N2 = NTOK * NTOK
NH = 16
NRD = (2 * WS - 1) ** 2 + 3
LANES = 16
F = NH * N2
C = 5125
NCH = N2 // C
KG = 321
S_FULL = KG * LANES
S_LAST = 5128
W = 5152
PADN = 1050656
NC = 2
NS = 16
NW = NC * NS
HHALF = NH // 2
NU = NCH * 2


@functools.partial(
    pl.kernel,
    out_type=jax.ShapeDtypeStruct((F,), jnp.float32),
    mesh=plsc.VectorSubcoreMesh(core_axis_name="c", subcore_axis_name="s"),
    compiler_params=pltpu.CompilerParams(needs_layout_passes=False),
    scratch_types=[
        pltpu.VMEM((NRD * NH,), jnp.float32),
        pltpu.VMEM((W,), jnp.int32),
        pltpu.VMEM((S_FULL,), jnp.float32),
    ],
)
def _rpb_kernel(table_hbm, idxp_hbm, out_hbm, table_v, win_v, out_v):
    wid = lax.axis_index("s") * NC + lax.axis_index("c")
    pltpu.sync_copy(table_hbm, table_v)

    def unit(i, carry):
        u = wid + i * NW
        c = u // 2
        h0 = (u % 2) * HHALF
        p0 = c * C
        qb = (p0 // 8) * 8
        pltpu.sync_copy(idxp_hbm.at[pl.ds(pl.multiple_of(qb, 8), W)], win_v)

        def head(hh, carry2):
            h = h0 + hh
            o = h * N2 + p0
            shift = lax.rem(h + p0, 8)
            w0 = o - shift
            off0 = p0 - qb + 8 - shift
            base = h * NRD

            t0 = lax.iota(jnp.int32, LANES)
            bv0 = jnp.maximum(
                base - NRD * (t0 < (shift - p0)).astype(jnp.int32), 0)
            rows0 = win_v[pl.ds(off0, LANES)]
            out_v[pl.ds(0, LANES)] = plsc.load_gather(table_v, [rows0 + bv0])

            def mid(k, carry3):
                rows = win_v[pl.ds(off0 + k * LANES, LANES)]
                out_v[pl.ds(k * LANES, LANES)] = plsc.load_gather(
                    table_v, [rows + base])
                return carry3

            lax.fori_loop(1, KG - 1, mid, 0, unroll=8)

            tl = (KG - 1) * LANES + t0
            bvl = jnp.minimum(
                base + NRD * (tl >= (N2 - p0 + shift)).astype(jnp.int32),
                (NH - 1) * NRD)
            rowsl = win_v[pl.ds(off0 + (KG - 1) * LANES, LANES)]
            out_v[pl.ds((KG - 1) * LANES, LANES)] = plsc.load_gather(
                table_v, [rowsl + bvl])

            is_last = jnp.logical_and(c == NCH - 1, h == NH - 1)

            @pl.when(jnp.logical_not(is_last))
            def _full():
                pltpu.sync_copy(
                    out_v.at[pl.ds(0, S_FULL)],
                    out_hbm.at[pl.ds(pl.multiple_of(w0, 8), S_FULL)])

            @pl.when(is_last)
            def _last():
                pltpu.sync_copy(
                    out_v.at[pl.ds(0, S_LAST)],
                    out_hbm.at[pl.ds(pl.multiple_of(w0, 8), S_LAST)])

            return carry2

        lax.fori_loop(0, HHALF, head, 0, unroll=False)
        return carry

    nu_w = (NU - wid + NW - 1) // NW
    lax.fori_loop(0, nu_w, unit, 0, unroll=False)


def kernel(relative_position_bias_table, relative_position_index):
    idx_flat = relative_position_index.reshape(-1)
    idx_pad = (
        jnp.zeros((PADN,), jnp.int32)
        .at[8:8 + N2].set(idx_flat)
        .at[0:8].set(idx_flat[N2 - 8:])
        .at[8 + N2:8 + N2 + 16].set(idx_flat[:16])
    )
    out = _rpb_kernel(relative_position_bias_table.T.reshape(-1), idx_pad)
    return out.reshape(NH, NTOK, NTOK)


# speedup vs baseline: 5.2386x; 1.0400x over previous
import functools

import jax
import jax.numpy as jnp
from jax import lax
from jax.experimental import pallas as pl
from jax.experimental.pallas import tpu as pltpu
from jax.experimental.pallas import tpu_sc as plsc

WS = 32
NTOK = WS * WS + 1
N2 = NTOK * NTOK
NH = 16
NRD = (2 * WS - 1) ** 2 + 3
LANES = 16
F = NH * N2
C = 5125
NCH = N2 // C
KG = 321
S_FULL = KG * LANES
S_LAST = 5128
W = 5152
PADN = 1050656
NC = 2
NS = 16
HHALF = NH // NC
HNRD = HHALF * NRD


@functools.partial(
    pl.kernel,
    out_type=jax.ShapeDtypeStruct((F,), jnp.float32),
    mesh=plsc.VectorSubcoreMesh(core_axis_name="c", subcore_axis_name="s"),
    compiler_params=pltpu.CompilerParams(needs_layout_passes=False),
    scratch_types=[
        pltpu.VMEM((HNRD,), jnp.float32),
        pltpu.VMEM((W,), jnp.int32),
        pltpu.VMEM((S_FULL,), jnp.float32),
        pltpu.VMEM((S_FULL,), jnp.float32),
        pltpu.SemaphoreType.DMA,
        pltpu.SemaphoreType.DMA,
    ],
)
def _rpb_kernel(table_hbm, idxp_hbm, out_hbm, table_v, win_v, out_a, out_b,
                sem_a, sem_b):
    core = lax.axis_index("c")
    sid = lax.axis_index("s")
    h0 = core * HHALF
    tb_off = h0 * NRD
    pltpu.sync_copy(table_hbm.at[pl.ds(pl.multiple_of(tb_off, 8), HNRD)],
                    table_v)
    bufs = (out_a, out_b)
    sems = (sem_a, sem_b)

    def unit(i, carry):
        c = sid + i * NS
        p0 = c * C
        qb = (p0 // 8) * 8
        pltpu.sync_copy(idxp_hbm.at[pl.ds(pl.multiple_of(qb, 8), W)], win_v)

        t0 = lax.iota(jnp.int32, LANES)
        tl = (KG - 1) * LANES + t0
        handles = [None, None]

        for hh in range(HHALF):
            h = h0 + hh
            o = h * N2 + p0
            shift = lax.rem(h + p0, 8)
            w0 = o - shift
            off0 = p0 - qb + 8 - shift
            base = hh * NRD
            ob = bufs[hh % 2]

            if handles[hh % 2] is not None:
                handles[hh % 2].wait()

            bv0 = jnp.maximum(
                base - NRD * (t0 < (shift - p0)).astype(jnp.int32), 0)
            rows0 = win_v[pl.ds(off0, LANES)]
            ob[pl.ds(0, LANES)] = plsc.load_gather(table_v, [rows0 + bv0])

            def mid(k, carry3, _ob=ob, _off0=off0, _base=base):
                rows = win_v[pl.ds(_off0 + k * LANES, LANES)]
                _ob[pl.ds(k * LANES, LANES)] = plsc.load_gather(
                    table_v, [rows + _base])
                return carry3

            lax.fori_loop(1, KG - 1, mid, 0, unroll=8)

            bvl = jnp.minimum(
                base + NRD * (tl >= (N2 - p0 + shift)).astype(jnp.int32),
                (HHALF - 1) * NRD)
            rowsl = win_v[pl.ds(off0 + (KG - 1) * LANES, LANES)]
            ob[pl.ds((KG - 1) * LANES, LANES)] = plsc.load_gather(
                table_v, [rowsl + bvl])

            if hh < HHALF - 1:
                handles[hh % 2] = pltpu.async_copy(
                    ob.at[pl.ds(0, S_FULL)],
                    out_hbm.at[pl.ds(pl.multiple_of(w0, 8), S_FULL)],
                    sems[hh % 2])
            else:
                is_last = c == NCH - 1

                @pl.when(jnp.logical_not(is_last))
                def _full():
                    pltpu.sync_copy(
                        ob.at[pl.ds(0, S_FULL)],
                        out_hbm.at[pl.ds(pl.multiple_of(w0, 8), S_FULL)])

                @pl.when(is_last)
                def _last():
                    pltpu.sync_copy(
                        ob.at[pl.ds(0, S_LAST)],
                        out_hbm.at[pl.ds(pl.multiple_of(w0, 8), S_LAST)])

        handles[(HHALF - 2) % 2].wait()
        return carry

    nu_w = (NCH - sid + NS - 1) // NS
    lax.fori_loop(0, nu_w, unit, 0, unroll=False)


def kernel(relative_position_bias_table, relative_position_index):
    idx_flat = relative_position_index.reshape(-1)
    idx_pad = (
        jnp.zeros((PADN,), jnp.int32)
        .at[8:8 + N2].set(idx_flat)
        .at[0:8].set(idx_flat[N2 - 8:])
        .at[8 + N2:8 + N2 + 16].set(idx_flat[:16])
    )
    out = _rpb_kernel(relative_position_bias_table.T.reshape(-1), idx_pad)
    return out.reshape(NH, NTOK, NTOK)


# speedup vs baseline: 5.4788x; 1.0458x over previous
import functools

import jax
import jax.numpy as jnp
from jax import lax
from jax.experimental import pallas as pl
from jax.experimental.pallas import tpu as pltpu
from jax.experimental.pallas import tpu_sc as plsc

WS = 32
NTOK = WS * WS + 1
N2 = NTOK * NTOK
NH = 16
NRD = (2 * WS - 1) ** 2 + 3
NRDP = NRD + 4
LANES = 16
F = NH * N2
C = 5125
NCH = N2 // C
KG = 321
S_FULL = KG * LANES
S_LAST = 5128
W = 5152
PADN = 1050656
NC = 2
NS = 16
HHALF = NH // NC
HNRD = HHALF * NRDP


@functools.partial(
    pl.kernel,
    out_type=jax.ShapeDtypeStruct((F,), jnp.float32),
    mesh=plsc.VectorSubcoreMesh(core_axis_name="c", subcore_axis_name="s"),
    compiler_params=pltpu.CompilerParams(needs_layout_passes=False),
    scratch_types=[
        pltpu.VMEM((HNRD,), jnp.float32),
        pltpu.VMEM((W,), jnp.int32),
        pltpu.VMEM((S_FULL,), jnp.float32),
        pltpu.VMEM((S_FULL,), jnp.float32),
        pltpu.SemaphoreType.DMA,
        pltpu.SemaphoreType.DMA,
    ],
)
def _rpb_kernel(table_hbm, idxp_hbm, out_hbm, table_v, win_v, out_a, out_b,
                sem_a, sem_b):
    core = lax.axis_index("c")
    sid = lax.axis_index("s")
    h0 = core * HHALF
    tb_off = h0 * NRDP
    pltpu.sync_copy(table_hbm.at[pl.ds(pl.multiple_of(tb_off, 8), HNRD)],
                    table_v)
    bufs = (out_a, out_b)
    sems = (sem_a, sem_b)

    def unit(i, carry):
        c = sid + i * NS
        p0 = c * C
        qb = (p0 // 8) * 8
        pltpu.sync_copy(idxp_hbm.at[pl.ds(pl.multiple_of(qb, 8), W)], win_v)

        t0 = lax.iota(jnp.int32, LANES)
        tl = (KG - 1) * LANES + t0
        handles = [None, None]

        for hh in range(HHALF):
            h = h0 + hh
            o = h * N2 + p0
            shift = lax.rem(h + p0, 8)
            w0 = o - shift
            off0 = p0 - qb + 8 - shift
            base = hh * NRDP
            ob = bufs[hh % 2]

            if handles[hh % 2] is not None:
                handles[hh % 2].wait()

            bv0 = jnp.maximum(
                base - NRDP * (t0 < (shift - p0)).astype(jnp.int32), 0)
            rows0 = win_v[pl.ds(off0, LANES)]
            ob[pl.ds(0, LANES)] = plsc.load_gather(table_v, [rows0 + bv0])

            tab_h = table_v.at[pl.ds(base, NRD)]

            def mid(k, carry3, _ob=ob, _off0=off0, _tab=tab_h):
                rows = win_v[pl.ds(_off0 + k * LANES, LANES)]
                _ob[pl.ds(k * LANES, LANES)] = plsc.load_gather(_tab, [rows])
                return carry3

            lax.fori_loop(1, KG - 1, mid, 0, unroll=8)

            bvl = jnp.minimum(
                base + NRDP * (tl >= (N2 - p0 + shift)).astype(jnp.int32),
                (HHALF - 1) * NRDP)
            rowsl = win_v[pl.ds(off0 + (KG - 1) * LANES, LANES)]
            ob[pl.ds((KG - 1) * LANES, LANES)] = plsc.load_gather(
                table_v, [rowsl + bvl])

            if hh < HHALF - 1:
                handles[hh % 2] = pltpu.async_copy(
                    ob.at[pl.ds(0, S_FULL)],
                    out_hbm.at[pl.ds(pl.multiple_of(w0, 8), S_FULL)],
                    sems[hh % 2])
            else:
                is_last = c == NCH - 1

                @pl.when(jnp.logical_not(is_last))
                def _full():
                    pltpu.sync_copy(
                        ob.at[pl.ds(0, S_FULL)],
                        out_hbm.at[pl.ds(pl.multiple_of(w0, 8), S_FULL)])

                @pl.when(is_last)
                def _last():
                    pltpu.sync_copy(
                        ob.at[pl.ds(0, S_LAST)],
                        out_hbm.at[pl.ds(pl.multiple_of(w0, 8), S_LAST)])

        handles[(HHALF - 2) % 2].wait()
        return carry

    nu_w = (NCH - sid + NS - 1) // NS
    lax.fori_loop(0, nu_w, unit, 0, unroll=False)


def kernel(relative_position_bias_table, relative_position_index):
    idx_flat = relative_position_index.reshape(-1)
    idx_pad = (
        jnp.zeros((PADN,), jnp.int32)
        .at[8:8 + N2].set(idx_flat)
        .at[0:8].set(idx_flat[N2 - 8:])
        .at[8 + N2:8 + N2 + 16].set(idx_flat[:16])
    )
    table_t = jnp.pad(relative_position_bias_table.T, ((0, 0), (0, NRDP - NRD)))
    out = _rpb_kernel(table_t.reshape(-1), idx_pad)
    return out.reshape(NH, NTOK, NTOK)


# speedup vs baseline: 5.4914x; 1.0023x over previous
import functools

import jax
import jax.numpy as jnp
from jax import lax
from jax.experimental import pallas as pl
from jax.experimental.pallas import tpu as pltpu
from jax.experimental.pallas import tpu_sc as plsc

WS = 32
NTOK = WS * WS + 1
N2 = NTOK * NTOK
NH = 16
NRD = (2 * WS - 1) ** 2 + 3
NRDP = NRD + 4
LANES = 16
F = NH * N2
C = 5125
NCH = N2 // C
KG = 321
S_FULL = KG * LANES
S_LAST = 5128
W = 5152
PADN = 1050656
NC = 2
NS = 16
HHALF = NH // NC
HNRD = HHALF * NRDP


@functools.partial(
    pl.kernel,
    out_type=jax.ShapeDtypeStruct((F,), jnp.float32),
    mesh=plsc.VectorSubcoreMesh(core_axis_name="c", subcore_axis_name="s"),
    compiler_params=pltpu.CompilerParams(needs_layout_passes=False),
    scratch_types=[
        pltpu.VMEM((HNRD,), jnp.float32),
        pltpu.VMEM((W,), jnp.int32),
        pltpu.VMEM((S_FULL,), jnp.float32),
        pltpu.VMEM((S_FULL,), jnp.float32),
        pltpu.SemaphoreType.DMA,
        pltpu.SemaphoreType.DMA,
    ],
)
def _rpb_kernel(table_hbm, idxp_hbm, out_hbm, table_v, win_v, out_a, out_b,
                sem_a, sem_b):
    core = lax.axis_index("c")
    sid = lax.axis_index("s")
    h0 = core * HHALF
    tb_off = h0 * NRDP
    pltpu.sync_copy(table_hbm.at[pl.ds(pl.multiple_of(tb_off, 8), HNRD)],
                    table_v)
    bufs = (out_a, out_b)
    sems = (sem_a, sem_b)

    def unit(i, carry):
        c = sid + i * NS
        p0 = c * C
        qb = (p0 // 8) * 8
        pltpu.sync_copy(idxp_hbm.at[pl.ds(pl.multiple_of(qb, 8), W)], win_v)

        t0 = lax.iota(jnp.int32, LANES)
        tl = (KG - 1) * LANES + t0
        handles = [None, None]

        for hh in range(HHALF):
            h = h0 + hh
            o = h * N2 + p0
            shift = lax.rem(h + p0, 8)
            w0 = o - shift
            off0 = p0 - qb + 8 - shift
            base = hh * NRDP
            ob = bufs[hh % 2]

            if handles[hh % 2] is not None:
                handles[hh % 2].wait()

            bv0 = jnp.maximum(
                base - NRDP * (t0 < (shift - p0)).astype(jnp.int32), 0)
            rows0 = win_v[pl.ds(off0, LANES)]
            ob[pl.ds(0, LANES)] = plsc.load_gather(table_v, [rows0 + bv0])

            tab_h = table_v.at[pl.ds(base, NRD)]

            def mid(k, carry3, _ob=ob, _off0=off0, _tab=tab_h):
                rows = win_v[pl.ds(_off0 + k * LANES, LANES)]
                _ob[pl.ds(k * LANES, LANES)] = plsc.load_gather(_tab, [rows])
                return carry3

            lax.fori_loop(1, KG - 1, mid, 0, unroll=16)

            bvl = jnp.minimum(
                base + NRDP * (tl >= (N2 - p0 + shift)).astype(jnp.int32),
                (HHALF - 1) * NRDP)
            rowsl = win_v[pl.ds(off0 + (KG - 1) * LANES, LANES)]
            ob[pl.ds((KG - 1) * LANES, LANES)] = plsc.load_gather(
                table_v, [rowsl + bvl])

            if hh < HHALF - 1:
                handles[hh % 2] = pltpu.async_copy(
                    ob.at[pl.ds(0, S_FULL)],
                    out_hbm.at[pl.ds(pl.multiple_of(w0, 8), S_FULL)],
                    sems[hh % 2])
            else:
                is_last = c == NCH - 1

                @pl.when(jnp.logical_not(is_last))
                def _full():
                    pltpu.sync_copy(
                        ob.at[pl.ds(0, S_FULL)],
                        out_hbm.at[pl.ds(pl.multiple_of(w0, 8), S_FULL)])

                @pl.when(is_last)
                def _last():
                    pltpu.sync_copy(
                        ob.at[pl.ds(0, S_LAST)],
                        out_hbm.at[pl.ds(pl.multiple_of(w0, 8), S_LAST)])

        handles[(HHALF - 2) % 2].wait()
        return carry

    nu_w = (NCH - sid + NS - 1) // NS
    lax.fori_loop(0, nu_w, unit, 0, unroll=False)


def kernel(relative_position_bias_table, relative_position_index):
    idx_flat = relative_position_index.reshape(-1)
    idx_pad = (
        jnp.zeros((PADN,), jnp.int32)
        .at[8:8 + N2].set(idx_flat)
        .at[0:8].set(idx_flat[N2 - 8:])
        .at[8 + N2:8 + N2 + 16].set(idx_flat[:16])
    )
    table_t = jnp.pad(relative_position_bias_table.T, ((0, 0), (0, NRDP - NRD)))
    out = _rpb_kernel(table_t.reshape(-1), idx_pad)
    return out.reshape(NH, NTOK, NTOK)


# speedup vs baseline: 7.0521x; 1.2842x over previous
import functools

import jax
import jax.numpy as jnp
from jax import lax
from jax.experimental import pallas as pl
from jax.experimental.pallas import tpu as pltpu
from jax.experimental.pallas import tpu_sc as plsc

WS = 32
NTOK = WS * WS + 1
N2 = NTOK * NTOK
NH = 16
NRD = (2 * WS - 1) ** 2 + 3
NRDP = NRD + 4
LANES = 16
F = NH * N2
C = 5125
NCH = N2 // C
KG = 321
S_FULL = KG * LANES
S_LAST = 5128
W = 5168
PADN = 1050672
NC = 2
NS = 16
HHALF = NH // NC
HNRD = HHALF * NRDP


@functools.partial(
    pl.kernel,
    out_type=jax.ShapeDtypeStruct((F,), jnp.float32),
    mesh=plsc.VectorSubcoreMesh(core_axis_name="c", subcore_axis_name="s"),
    compiler_params=pltpu.CompilerParams(needs_layout_passes=False),
    scratch_types=[
        pltpu.VMEM((HNRD,), jnp.float32),
        pltpu.VMEM((W,), jnp.int32),
        pltpu.VMEM((S_FULL,), jnp.float32),
        pltpu.VMEM((S_FULL,), jnp.float32),
        pltpu.SemaphoreType.DMA,
        pltpu.SemaphoreType.DMA,
    ],
)
def _rpb_kernel(table_hbm, idxp_hbm, out_hbm, table_v, win_v, out_a, out_b,
                sem_a, sem_b):
    core = lax.axis_index("c")
    sid = lax.axis_index("s")
    h0 = core * HHALF
    tb_off = h0 * NRDP
    pltpu.sync_copy(table_hbm.at[pl.ds(pl.multiple_of(tb_off, 8), HNRD)],
                    table_v)
    bufs = (out_a, out_b)
    sems = (sem_a, sem_b)

    def unit(i, carry):
        c = sid + i * NS
        p0 = c * C
        qb = (p0 // 8) * 8
        pltpu.sync_copy(idxp_hbm.at[pl.ds(pl.multiple_of(qb, 8), W)], win_v)

        t0 = lax.iota(jnp.int32, LANES)
        tl = (KG - 1) * LANES + t0
        handles = [None, None]

        for hh in range(HHALF):
            h = h0 + hh
            o = h * N2 + p0
            shift = lax.rem(h + p0, 8)
            w0 = o - shift
            off0 = p0 - qb + 8 - shift
            base = hh * NRDP
            ob = bufs[hh % 2]

            if handles[hh % 2] is not None:
                handles[hh % 2].wait()

            tab_h = table_v.at[pl.ds(base, NRD)]

            def mid(k, carry3, _ob=ob, _off0=off0, _tab=tab_h):
                rows_r, vals_r = carry3
                rows_n = win_v[pl.ds(_off0 + (k + 1) * LANES, LANES)]
                vals_n = plsc.load_gather(_tab, [rows_r])
                _ob[pl.ds((k - 1) * LANES, LANES)] = vals_r
                return (rows_n, vals_n)

            rows0 = win_v[pl.ds(off0, LANES)]
            rows1 = win_v[pl.ds(off0 + LANES, LANES)]
            vals0 = plsc.load_gather(tab_h, [rows0])
            _, vals_f = lax.fori_loop(1, KG, mid, (rows1, vals0), unroll=8)
            ob[pl.ds((KG - 1) * LANES, LANES)] = vals_f

            @pl.when(p0 == 0)
            def _front(_ob=ob, _off0=off0, _base=base, _shift=shift):
                bv0 = jnp.maximum(
                    _base - NRDP * (t0 < _shift).astype(jnp.int32), 0)
                rws = win_v[pl.ds(_off0, LANES)]
                _ob[pl.ds(0, LANES)] = plsc.load_gather(table_v, [rws + bv0])

            @pl.when(c == NCH - 1)
            def _end(_ob=ob, _off0=off0, _base=base, _shift=shift, _p0=p0):
                bvl = jnp.minimum(
                    _base
                    + NRDP * (tl >= (N2 - _p0 + _shift)).astype(jnp.int32),
                    (HHALF - 1) * NRDP)
                rws = win_v[pl.ds(_off0 + (KG - 1) * LANES, LANES)]
                _ob[pl.ds((KG - 1) * LANES, LANES)] = plsc.load_gather(
                    table_v, [rws + bvl])

            if hh < HHALF - 1:
                handles[hh % 2] = pltpu.async_copy(
                    ob.at[pl.ds(0, S_FULL)],
                    out_hbm.at[pl.ds(pl.multiple_of(w0, 8), S_FULL)],
                    sems[hh % 2])
            else:
                is_last = c == NCH - 1

                @pl.when(jnp.logical_not(is_last))
                def _full():
                    pltpu.sync_copy(
                        ob.at[pl.ds(0, S_FULL)],
                        out_hbm.at[pl.ds(pl.multiple_of(w0, 8), S_FULL)])

                @pl.when(is_last)
                def _last():
                    pltpu.sync_copy(
                        ob.at[pl.ds(0, S_LAST)],
                        out_hbm.at[pl.ds(pl.multiple_of(w0, 8), S_LAST)])

        handles[(HHALF - 2) % 2].wait()
        return carry

    nu_w = (NCH - sid + NS - 1) // NS
    lax.fori_loop(0, nu_w, unit, 0, unroll=False)


def kernel(relative_position_bias_table, relative_position_index):
    idx_flat = relative_position_index.reshape(-1)
    idx_pad = (
        jnp.zeros((PADN,), jnp.int32)
        .at[8:8 + N2].set(idx_flat)
        .at[0:8].set(idx_flat[N2 - 8:])
        .at[8 + N2:8 + N2 + 16].set(idx_flat[:16])
    )
    table_t = jnp.pad(relative_position_bias_table.T, ((0, 0), (0, NRDP - NRD)))
    out = _rpb_kernel(table_t.reshape(-1), idx_pad)
    return out.reshape(NH, NTOK, NTOK)


# speedup vs baseline: 7.1023x; 1.0071x over previous
import functools

import jax
import jax.numpy as jnp
from jax import lax
from jax.experimental import pallas as pl
from jax.experimental.pallas import tpu as pltpu
from jax.experimental.pallas import tpu_sc as plsc

WS = 32
NTOK = WS * WS + 1
N2 = NTOK * NTOK
NH = 16
NRD = (2 * WS - 1) ** 2 + 3
NRDP = NRD + 4
LANES = 16
F = NH * N2
C = 5125
NCH = N2 // C
KG = 321
S_FULL = KG * LANES
S_LAST = 5128
W = 5168
PADN = 1050672
NC = 2
NS = 16
HHALF = NH // NC
HNRD = HHALF * NRDP


@functools.partial(
    pl.kernel,
    out_type=jax.ShapeDtypeStruct((F,), jnp.float32),
    mesh=plsc.VectorSubcoreMesh(core_axis_name="c", subcore_axis_name="s"),
    compiler_params=pltpu.CompilerParams(needs_layout_passes=False),
    scratch_types=[
        pltpu.VMEM((HNRD,), jnp.float32),
        pltpu.VMEM((W,), jnp.int32),
        pltpu.VMEM((S_FULL,), jnp.float32),
        pltpu.VMEM((S_FULL,), jnp.float32),
        pltpu.SemaphoreType.DMA,
        pltpu.SemaphoreType.DMA,
    ],
)
def _rpb_kernel(table_hbm, idxp_hbm, out_hbm, table_v, win_v, out_a, out_b,
                sem_a, sem_b):
    core = lax.axis_index("c")
    sid = lax.axis_index("s")
    h0 = core * HHALF
    tb_off = h0 * NRDP
    pltpu.sync_copy(table_hbm.at[pl.ds(pl.multiple_of(tb_off, 8), HNRD)],
                    table_v)
    bufs = (out_a, out_b)
    sems = (sem_a, sem_b)

    def unit(i, carry):
        c = sid + i * NS
        p0 = c * C
        qb = (p0 // 8) * 8
        pltpu.sync_copy(idxp_hbm.at[pl.ds(pl.multiple_of(qb, 8), W)], win_v)

        t0 = lax.iota(jnp.int32, LANES)
        tl = (KG - 1) * LANES + t0
        handles = [None, None]

        for hh in range(HHALF):
            h = h0 + hh
            o = h * N2 + p0
            shift = lax.rem(h + p0, 8)
            w0 = o - shift
            off0 = p0 - qb + 8 - shift
            base = hh * NRDP
            ob = bufs[hh % 2]

            if handles[hh % 2] is not None:
                handles[hh % 2].wait()

            tab_h = table_v.at[pl.ds(base, NRD)]

            def mid(k, carry3, _ob=ob, _off0=off0, _tab=tab_h):
                rows_r, vals_r = carry3
                rows_n = win_v[pl.ds(_off0 + (k + 1) * LANES, LANES)]
                vals_n = plsc.load_gather(_tab, [rows_r])
                _ob[pl.ds((k - 1) * LANES, LANES)] = vals_r
                return (rows_n, vals_n)

            rows0 = win_v[pl.ds(off0, LANES)]
            rows1 = win_v[pl.ds(off0 + LANES, LANES)]
            vals0 = plsc.load_gather(tab_h, [rows0])
            _, vals_f = lax.fori_loop(1, KG, mid, (rows1, vals0), unroll=16)
            ob[pl.ds((KG - 1) * LANES, LANES)] = vals_f

            @pl.when(p0 == 0)
            def _front(_ob=ob, _off0=off0, _base=base, _shift=shift):
                bv0 = jnp.maximum(
                    _base - NRDP * (t0 < _shift).astype(jnp.int32), 0)
                rws = win_v[pl.ds(_off0, LANES)]
                _ob[pl.ds(0, LANES)] = plsc.load_gather(table_v, [rws + bv0])

            @pl.when(c == NCH - 1)
            def _end(_ob=ob, _off0=off0, _base=base, _shift=shift, _p0=p0):
                bvl = jnp.minimum(
                    _base
                    + NRDP * (tl >= (N2 - _p0 + _shift)).astype(jnp.int32),
                    (HHALF - 1) * NRDP)
                rws = win_v[pl.ds(_off0 + (KG - 1) * LANES, LANES)]
                _ob[pl.ds((KG - 1) * LANES, LANES)] = plsc.load_gather(
                    table_v, [rws + bvl])

            if hh < HHALF - 1:
                handles[hh % 2] = pltpu.async_copy(
                    ob.at[pl.ds(0, S_FULL)],
                    out_hbm.at[pl.ds(pl.multiple_of(w0, 8), S_FULL)],
                    sems[hh % 2])
            else:
                is_last = c == NCH - 1

                @pl.when(jnp.logical_not(is_last))
                def _full():
                    pltpu.sync_copy(
                        ob.at[pl.ds(0, S_FULL)],
                        out_hbm.at[pl.ds(pl.multiple_of(w0, 8), S_FULL)])

                @pl.when(is_last)
                def _last():
                    pltpu.sync_copy(
                        ob.at[pl.ds(0, S_LAST)],
                        out_hbm.at[pl.ds(pl.multiple_of(w0, 8), S_LAST)])

        handles[(HHALF - 2) % 2].wait()
        return carry

    nu_w = (NCH - sid + NS - 1) // NS
    lax.fori_loop(0, nu_w, unit, 0, unroll=False)


def kernel(relative_position_bias_table, relative_position_index):
    idx_flat = relative_position_index.reshape(-1)
    idx_pad = (
        jnp.zeros((PADN,), jnp.int32)
        .at[8:8 + N2].set(idx_flat)
        .at[0:8].set(idx_flat[N2 - 8:])
        .at[8 + N2:8 + N2 + 16].set(idx_flat[:16])
    )
    table_t = jnp.pad(relative_position_bias_table.T, ((0, 0), (0, NRDP - NRD)))
    out = _rpb_kernel(table_t.reshape(-1), idx_pad)
    return out.reshape(NH, NTOK, NTOK)


# speedup vs baseline: 7.3650x; 1.0370x over previous
import functools

import jax
import jax.numpy as jnp
from jax import lax
from jax.experimental import pallas as pl
from jax.experimental.pallas import tpu as pltpu
from jax.experimental.pallas import tpu_sc as plsc

WS = 32
NTOK = WS * WS + 1
N2 = NTOK * NTOK
NH = 16
NRD = (2 * WS - 1) ** 2 + 3
NRDP = NRD + 4
LANES = 16
F = NH * N2
C = 5125
NCH = N2 // C
KG = 321
S_FULL = KG * LANES
S_LAST = 5128
W = 5184
PADN = 1050680
NC = 2
NS = 16
HHALF = NH // NC
HNRD = HHALF * NRDP


@functools.partial(
    pl.kernel,
    out_type=jax.ShapeDtypeStruct((F,), jnp.float32),
    mesh=plsc.VectorSubcoreMesh(core_axis_name="c", subcore_axis_name="s"),
    compiler_params=pltpu.CompilerParams(needs_layout_passes=False),
    scratch_types=[
        pltpu.VMEM((HNRD,), jnp.float32),
        pltpu.VMEM((W,), jnp.int32),
        pltpu.VMEM((S_FULL,), jnp.float32),
        pltpu.VMEM((S_FULL,), jnp.float32),
        pltpu.SemaphoreType.DMA,
        pltpu.SemaphoreType.DMA,
    ],
)
def _rpb_kernel(table_hbm, idxp_hbm, out_hbm, table_v, win_v, out_a, out_b,
                sem_a, sem_b):
    core = lax.axis_index("c")
    sid = lax.axis_index("s")
    h0 = core * HHALF
    tb_off = h0 * NRDP
    pltpu.sync_copy(table_hbm.at[pl.ds(pl.multiple_of(tb_off, 8), HNRD)],
                    table_v)
    bufs = (out_a, out_b)
    sems = (sem_a, sem_b)

    def unit(i, carry):
        c = sid + i * NS
        p0 = c * C
        qb = (p0 // 8) * 8
        pltpu.sync_copy(idxp_hbm.at[pl.ds(pl.multiple_of(qb, 8), W)], win_v)

        t0 = lax.iota(jnp.int32, LANES)
        tl = (KG - 1) * LANES + t0
        handles = [None, None]

        for hh in range(HHALF):
            h = h0 + hh
            o = h * N2 + p0
            shift = lax.rem(h + p0, 8)
            w0 = o - shift
            off0 = p0 - qb + 8 - shift
            base = hh * NRDP
            ob = bufs[hh % 2]

            if handles[hh % 2] is not None:
                handles[hh % 2].wait()

            tab_h = table_v.at[pl.ds(base, NRD)]

            def mid(k, carry3, _ob=ob, _off0=off0, _tab=tab_h):
                ra, rb, vals_r = carry3
                rows_n = win_v[pl.ds(_off0 + (k + 2) * LANES, LANES)]
                vals_n = plsc.load_gather(_tab, [ra])
                _ob[pl.ds((k - 1) * LANES, LANES)] = vals_r
                return (rb, rows_n, vals_n)

            rows0 = win_v[pl.ds(off0, LANES)]
            rows1 = win_v[pl.ds(off0 + LANES, LANES)]
            rows2 = win_v[pl.ds(off0 + 2 * LANES, LANES)]
            vals0 = plsc.load_gather(tab_h, [rows0])
            _, _, vals_f = lax.fori_loop(1, KG, mid, (rows1, rows2, vals0),
                                         unroll=16)
            ob[pl.ds((KG - 1) * LANES, LANES)] = vals_f

            @pl.when(p0 == 0)
            def _front(_ob=ob, _off0=off0, _base=base, _shift=shift):
                bv0 = jnp.maximum(
                    _base - NRDP * (t0 < _shift).astype(jnp.int32), 0)
                rws = win_v[pl.ds(_off0, LANES)]
                _ob[pl.ds(0, LANES)] = plsc.load_gather(table_v, [rws + bv0])

            @pl.when(c == NCH - 1)
            def _end(_ob=ob, _off0=off0, _base=base, _shift=shift, _p0=p0):
                bvl = jnp.minimum(
                    _base
                    + NRDP * (tl >= (N2 - _p0 + _shift)).astype(jnp.int32),
                    (HHALF - 1) * NRDP)
                rws = win_v[pl.ds(_off0 + (KG - 1) * LANES, LANES)]
                _ob[pl.ds((KG - 1) * LANES, LANES)] = plsc.load_gather(
                    table_v, [rws + bvl])

            if hh < HHALF - 1:
                handles[hh % 2] = pltpu.async_copy(
                    ob.at[pl.ds(0, S_FULL)],
                    out_hbm.at[pl.ds(pl.multiple_of(w0, 8), S_FULL)],
                    sems[hh % 2])
            else:
                is_last = c == NCH - 1

                @pl.when(jnp.logical_not(is_last))
                def _full():
                    pltpu.sync_copy(
                        ob.at[pl.ds(0, S_FULL)],
                        out_hbm.at[pl.ds(pl.multiple_of(w0, 8), S_FULL)])

                @pl.when(is_last)
                def _last():
                    pltpu.sync_copy(
                        ob.at[pl.ds(0, S_LAST)],
                        out_hbm.at[pl.ds(pl.multiple_of(w0, 8), S_LAST)])

        handles[(HHALF - 2) % 2].wait()
        return carry

    nu_w = (NCH - sid + NS - 1) // NS
    lax.fori_loop(0, nu_w, unit, 0, unroll=False)


def kernel(relative_position_bias_table, relative_position_index):
    idx_flat = relative_position_index.reshape(-1)
    idx_pad = (
        jnp.zeros((PADN,), jnp.int32)
        .at[8:8 + N2].set(idx_flat)
        .at[0:8].set(idx_flat[N2 - 8:])
        .at[8 + N2:8 + N2 + 16].set(idx_flat[:16])
    )
    table_t = jnp.pad(relative_position_bias_table.T, ((0, 0), (0, NRDP - NRD)))
    out = _rpb_kernel(table_t.reshape(-1), idx_pad)
    return out.reshape(NH, NTOK, NTOK)


# speedup vs baseline: 7.3813x; 1.0022x over previous
import functools

import jax
import jax.numpy as jnp
from jax import lax
from jax.experimental import pallas as pl
from jax.experimental.pallas import tpu as pltpu
from jax.experimental.pallas import tpu_sc as plsc

WS = 32
NTOK = WS * WS + 1
N2 = NTOK * NTOK
NH = 16
NRD = (2 * WS - 1) ** 2 + 3
NRDP = NRD + 4
LANES = 16
F = NH * N2
C = 5125
NCH = N2 // C
KG = 321
S_FULL = KG * LANES
S_LAST = 5128
W = 5184
PADN = 1050680
NC = 2
NS = 16
HHALF = NH // NC
HNRD = HHALF * NRDP


@functools.partial(
    pl.kernel,
    out_type=jax.ShapeDtypeStruct((F,), jnp.float32),
    mesh=plsc.VectorSubcoreMesh(core_axis_name="c", subcore_axis_name="s"),
    compiler_params=pltpu.CompilerParams(needs_layout_passes=False),
    scratch_types=[
        pltpu.VMEM((HNRD,), jnp.float32),
        pltpu.VMEM((W,), jnp.int32),
        pltpu.VMEM((S_FULL,), jnp.float32),
        pltpu.VMEM((S_FULL,), jnp.float32),
        pltpu.SemaphoreType.DMA,
        pltpu.SemaphoreType.DMA,
    ],
)
def _rpb_kernel(table_hbm, idxp_hbm, out_hbm, table_v, win_v, out_a, out_b,
                sem_a, sem_b):
    core = lax.axis_index("c")
    sid = lax.axis_index("s")
    h0 = core * HHALF
    tb_off = h0 * NRDP
    pltpu.sync_copy(table_hbm.at[pl.ds(pl.multiple_of(tb_off, 8), HNRD)],
                    table_v)
    bufs = (out_a, out_b)
    sems = (sem_a, sem_b)

    def unit(i, carry):
        c = sid + i * NS
        p0 = c * C
        qb = (p0 // 8) * 8
        pltpu.sync_copy(idxp_hbm.at[pl.ds(pl.multiple_of(qb, 8), W)], win_v)

        t0 = lax.iota(jnp.int32, LANES)
        tl = (KG - 1) * LANES + t0
        handles = [None, None]

        for hh in range(HHALF):
            h = h0 + hh
            o = h * N2 + p0
            shift = lax.rem(h + p0, 8)
            w0 = o - shift
            off0 = p0 - qb + 8 - shift
            base = hh * NRDP
            ob = bufs[hh % 2]

            if handles[hh % 2] is not None:
                handles[hh % 2].wait()

            tab_h = table_v.at[pl.ds(base, NRD)]

            def mid(k, carry3, _ob=ob, _off0=off0, _tab=tab_h):
                ra, rb, vals_r = carry3
                rows_n = win_v[pl.ds(_off0 + (k + 2) * LANES, LANES)]
                vals_n = plsc.load_gather(_tab, [ra])
                _ob[pl.ds((k - 1) * LANES, LANES)] = vals_r
                return (rb, rows_n, vals_n)

            rows0 = win_v[pl.ds(off0, LANES)]
            rows1 = win_v[pl.ds(off0 + LANES, LANES)]
            rows2 = win_v[pl.ds(off0 + 2 * LANES, LANES)]
            vals0 = plsc.load_gather(tab_h, [rows0])
            _, _, vals_f = lax.fori_loop(1, KG, mid, (rows1, rows2, vals0),
                                         unroll=32)
            ob[pl.ds((KG - 1) * LANES, LANES)] = vals_f

            @pl.when(p0 == 0)
            def _front(_ob=ob, _off0=off0, _base=base, _shift=shift):
                bv0 = jnp.maximum(
                    _base - NRDP * (t0 < _shift).astype(jnp.int32), 0)
                rws = win_v[pl.ds(_off0, LANES)]
                _ob[pl.ds(0, LANES)] = plsc.load_gather(table_v, [rws + bv0])

            @pl.when(c == NCH - 1)
            def _end(_ob=ob, _off0=off0, _base=base, _shift=shift, _p0=p0):
                bvl = jnp.minimum(
                    _base
                    + NRDP * (tl >= (N2 - _p0 + _shift)).astype(jnp.int32),
                    (HHALF - 1) * NRDP)
                rws = win_v[pl.ds(_off0 + (KG - 1) * LANES, LANES)]
                _ob[pl.ds((KG - 1) * LANES, LANES)] = plsc.load_gather(
                    table_v, [rws + bvl])

            if hh < HHALF - 1:
                handles[hh % 2] = pltpu.async_copy(
                    ob.at[pl.ds(0, S_FULL)],
                    out_hbm.at[pl.ds(pl.multiple_of(w0, 8), S_FULL)],
                    sems[hh % 2])
            else:
                is_last = c == NCH - 1

                @pl.when(jnp.logical_not(is_last))
                def _full():
                    pltpu.sync_copy(
                        ob.at[pl.ds(0, S_FULL)],
                        out_hbm.at[pl.ds(pl.multiple_of(w0, 8), S_FULL)])

                @pl.when(is_last)
                def _last():
                    pltpu.sync_copy(
                        ob.at[pl.ds(0, S_LAST)],
                        out_hbm.at[pl.ds(pl.multiple_of(w0, 8), S_LAST)])

        handles[(HHALF - 2) % 2].wait()
        return carry

    nu_w = (NCH - sid + NS - 1) // NS
    lax.fori_loop(0, nu_w, unit, 0, unroll=False)


def kernel(relative_position_bias_table, relative_position_index):
    idx_flat = relative_position_index.reshape(-1)
    idx_pad = (
        jnp.zeros((PADN,), jnp.int32)
        .at[8:8 + N2].set(idx_flat)
        .at[0:8].set(idx_flat[N2 - 8:])
        .at[8 + N2:8 + N2 + 16].set(idx_flat[:16])
    )
    table_t = jnp.pad(relative_position_bias_table.T, ((0, 0), (0, NRDP - NRD)))
    out = _rpb_kernel(table_t.reshape(-1), idx_pad)
    return out.reshape(NH, NTOK, NTOK)
